# baseline (speedup 1.0000x reference)
import functools

import jax
import jax.numpy as jnp
from jax import lax
from jax.experimental import pallas as pl
from jax.experimental.pallas import tpu as pltpu
from jax.experimental.pallas import tpu_sc as plsc

N = 10000
E = 160000
D = 256
H = 128
NC = 2
NS = 16
K = 128
E_PAD = 163840
R_SH = 10112

_EPT_S = E_PAD // NS
_EPT_D = E_PAD // (NC * NS)

_mesh = plsc.VectorSubcoreMesh(
    core_axis_name="c", subcore_axis_name="s", num_cores=NC, num_subcores=NS)

def _memset2d(ref, rows, cols):
    zero16 = jnp.zeros((16,), jnp.float32)

    def body(i, _):
        for j in range(cols // 16):
            ref[i, pl.ds(j * 16, 16)] = zero16
        return 0
    lax.fori_loop(0, rows, body, 0, unroll=False)


def _zero_shared_slice(zbuf, shared, base, rows, bufrows=K):
    full, rem = rows // bufrows, rows % bufrows
    for i in range(full):
        pltpu.sync_copy(zbuf, shared.at[pl.ds(base + i * bufrows, bufrows)])
    if rem:
        pltpu.sync_copy(zbuf.at[pl.ds(0, rem)],
                        shared.at[pl.ds(base + full * bufrows, rem)])


@functools.partial(
    pl.kernel,
    out_type=[jax.ShapeDtypeStruct((N, H), jnp.float32),
              jax.ShapeDtypeStruct((N, H), jnp.float32)],
    mesh=_mesh,
    scratch_types=[
        pltpu.VMEM((K,), jnp.int32),
        pltpu.VMEM((K, H), jnp.float32),
        pltpu.VMEM_SHARED((R_SH, H), jnp.float32),
    ],
)
def _deg_kernel(dst_hbm, da_out, db_out, dst_v, ones_v, deg_sh):
    c = lax.axis_index("c")
    s = lax.axis_index("s")
    wid = s * NC + c

    _memset2d(ones_v, K, H)
    _zero_shared_slice(ones_v, deg_sh, s * (R_SH // NS), R_SH // NS)
    plsc.subcore_barrier()

    e0 = jnp.where(lax.iota(jnp.int32, 16) == 0,
                   jnp.float32(1.0), jnp.float32(0.0))

    def init_body(i, _):
        ones_v[i, pl.ds(0, 16)] = e0
        return 0
    lax.fori_loop(0, K, init_body, 0, unroll=False)

    def chunk(k, _):
        off = wid * _EPT_D + k * K
        pltpu.sync_copy(dst_hbm.at[pl.ds(off, K)], dst_v)
        pltpu.sync_copy(ones_v, deg_sh.at[dst_v], add=True)
        return 0
    lax.fori_loop(0, _EPT_D // K, chunk, 0, unroll=False)

    plsc.subcore_barrier()
    span = R_SH // NS
    last = N - span * (NS - 1)

    def copy_out(out_ref):
        @pl.when(s < NS - 1)
        def _():
            pltpu.sync_copy(deg_sh.at[pl.ds(s * span, span)],
                            out_ref.at[pl.ds(s * span, span)])

        @pl.when(s == NS - 1)
        def _():
            pltpu.sync_copy(deg_sh.at[pl.ds((NS - 1) * span, last)],
                            out_ref.at[pl.ds((NS - 1) * span, last)])

    @pl.when(c == 0)
    def _():
        copy_out(da_out)

    @pl.when(c == 1)
    def _():
        copy_out(db_out)


_CK = 112
_CHS = 92
_EPT = _CK * _CHS
_E_SEG = _EPT * NS
_PH = 4
_CPP = _CHS // _PH
_IPP = _CPP * _CK
_NIT = (_CPP + 2) // 3


@functools.partial(
    pl.kernel,
    out_type=[jax.ShapeDtypeStruct((N, H), jnp.float32),
              jax.ShapeDtypeStruct((N, H), jnp.float32)],
    mesh=_mesh,
    scratch_types=[
        pltpu.VMEM((_IPP,), jnp.int32),
        pltpu.VMEM((_IPP,), jnp.int32),
        pltpu.VMEM((_CK, H), jnp.float32),
        pltpu.VMEM((_CK, H), jnp.float32),
        pltpu.VMEM((_CK, H), jnp.float32),
        pltpu.VMEM_SHARED((R_SH, H), jnp.float32),
        pltpu.SemaphoreType.DMA,
        pltpu.SemaphoreType.DMA,
        pltpu.SemaphoreType.DMA,
    ],
)
def _seg_kernel(ya_hbm, yb_hbm, src_hbm, dst_hbm, sa_out, sb_out,
                srcv, dstv, r0, r1, r2, s_sh, sem0, sem1, sem2):
    c = lax.axis_index("c")
    s = lax.axis_index("s")

    bufs = [(r0, sem0), (r1, sem1), (r2, sem2)]

    _memset2d(r0, _CK, H)
    _zero_shared_slice(r0, s_sh, s * (R_SH // NS), R_SH // NS, bufrows=_CK)
    plsc.subcore_barrier()

    def main(y_hbm):
        def gather(k, rows, sem):
            pltpu.async_copy(
                y_hbm.at[srcv.at[pl.ds(k * _CK, _CK)]], rows, sem)

        def gwait(rows, sem):
            pltpu.make_async_copy(
                y_hbm.at[srcv.at[pl.ds(0, _CK)]], rows, sem).wait()

        def scat(k, rows):
            pltpu.sync_copy(rows, s_sh.at[dstv.at[pl.ds(k * _CK, _CK)]],
                            add=True)

        for phase in range(_PH):
            off = s * _EPT + phase * _IPP
            pltpu.sync_copy(src_hbm.at[pl.ds(off, _IPP)], srcv)
            pltpu.sync_copy(dst_hbm.at[pl.ds(off, _IPP)], dstv)

            gather(0, *bufs[0])
            gather(1, *bufs[1])

            def body(i, _):
                for q in range(3):
                    k = i * 3 + q
                    cur = bufs[q]
                    nxt = bufs[(q + 2) % 3]

                    @pl.when(k + 2 < _CPP)
                    def _():
                        gather(k + 2, *nxt)

                    @pl.when(k < _CPP)
                    def _():
                        gwait(*cur)
                        scat(k, cur[0])
                return 0
            lax.fori_loop(0, _NIT, body, 0, unroll=False)

    @pl.when(c == 0)
    def _():
        main(ya_hbm)

    @pl.when(c == 1)
    def _():
        main(yb_hbm)

    plsc.subcore_barrier()

    span = R_SH // NS
    last = N - span * (NS - 1)

    def copy_out(out_ref):
        @pl.when(s < NS - 1)
        def _():
            pltpu.sync_copy(s_sh.at[pl.ds(s * span, span)],
                            out_ref.at[pl.ds(s * span, span)])

        @pl.when(s == NS - 1)
        def _():
            pltpu.sync_copy(s_sh.at[pl.ds((NS - 1) * span, last)],
                            out_ref.at[pl.ds((NS - 1) * span, last)])

    @pl.when(c == 0)
    def _():
        copy_out(sa_out)

    @pl.when(c == 1)
    def _():
        copy_out(sb_out)


_RB = 1000
_GRID = N // _RB


def _dinv_of(da_blk, db_blk):
    deg = jnp.sum(da_blk, axis=1) + jnp.sum(db_blk, axis=1) + 1.0
    return lax.rsqrt(deg)[:, None]


def _p1_body(x_ref, w_ref, b_ref, da_ref, db_ref, ya_ref, yb_ref, r_ref):
    dinv = _dinv_of(da_ref[...], db_ref[...])
    x = x_ref[...]
    y = jnp.dot(x, w_ref[...], preferred_element_type=jnp.float32)
    yp = y * dinv
    ya_ref[...] = yp[:, :H]
    yb_ref[...] = yp[:, H:]
    r_ref[...] = y * (dinv * dinv) + b_ref[...] + x


def _p2_body(sa_ref, sb_ref, r1_ref, x_ref, w_ref, b_ref, da_ref, db_ref,
             ya_ref, yb_ref, r_ref):
    dinv = _dinv_of(da_ref[...], db_ref[...])
    h = jnp.concatenate([sa_ref[...] * dinv, sb_ref[...] * dinv],
                        axis=1) + r1_ref[...]
    y = jnp.dot(h, w_ref[...], preferred_element_type=jnp.float32)
    yp = y * dinv
    ya_ref[...] = yp[:, :H]
    yb_ref[...] = yp[:, H:]
    r_ref[...] = y * (dinv * dinv) + b_ref[...] + x_ref[...]


def _p3_body(sa_ref, sb_ref, r2_ref, da_ref, db_ref, o_ref):
    dinv = _dinv_of(da_ref[...], db_ref[...])
    o_ref[...] = jnp.concatenate([sa_ref[...] * dinv, sb_ref[...] * dinv],
                                 axis=1) + r2_ref[...]


def _row_spec(w):
    return pl.BlockSpec((_RB, w), lambda i: (i, 0))


_FULL_W = pl.BlockSpec((D, D), lambda i: (0, 0))
_FULL_B = pl.BlockSpec((1, D), lambda i: (0, 0))


def _p1(x, W1, b1r, da, db):
    return pl.pallas_call(
        _p1_body, grid=(_GRID,),
        in_specs=[_row_spec(D), _FULL_W, _FULL_B, _row_spec(H), _row_spec(H)],
        out_specs=[_row_spec(H), _row_spec(H), _row_spec(D)],
        out_shape=[jax.ShapeDtypeStruct((N, H), jnp.float32),
                   jax.ShapeDtypeStruct((N, H), jnp.float32),
                   jax.ShapeDtypeStruct((N, D), jnp.float32)],
    )(x, W1, b1r, da, db)


def _p2(sa, sb, r1, x, W2, b2r, da, db):
    return pl.pallas_call(
        _p2_body, grid=(_GRID,),
        in_specs=[_row_spec(H), _row_spec(H), _row_spec(D), _row_spec(D),
                  _FULL_W, _FULL_B, _row_spec(H), _row_spec(H)],
        out_specs=[_row_spec(H), _row_spec(H), _row_spec(D)],
        out_shape=[jax.ShapeDtypeStruct((N, H), jnp.float32),
                   jax.ShapeDtypeStruct((N, H), jnp.float32),
                   jax.ShapeDtypeStruct((N, D), jnp.float32)],
    )(sa, sb, r1, x, W2, b2r, da, db)


def _p3(sa, sb, r2, da, db):
    return pl.pallas_call(
        _p3_body, grid=(_GRID,),
        in_specs=[_row_spec(H), _row_spec(H), _row_spec(D),
                  _row_spec(H), _row_spec(H)],
        out_specs=_row_spec(D),
        out_shape=jax.ShapeDtypeStruct((N, D), jnp.float32),
    )(sa, sb, r2, da, db)


def kernel(x, edges, W1, b1, W2, b2):
    e = edges.astype(jnp.int32)
    dst_d = jnp.concatenate([e[:, 1], jnp.full((E_PAD - E,), N, jnp.int32)])
    src_s = jnp.concatenate([e[:, 0], jnp.zeros((_E_SEG - E,), jnp.int32)])
    dst_s = jnp.concatenate([e[:, 1], jnp.full((_E_SEG - E,), N, jnp.int32)])

    da, db = _deg_kernel(dst_d)
    ya, yb, r1 = _p1(x, W1, b1.reshape(1, D), da, db)
    sa, sb = _seg_kernel(ya, yb, src_s, dst_s)
    y2a, y2b, r2 = _p2(sa, sb, r1, x, W2, b2.reshape(1, D), da, db)
    s2a, s2b = _seg_kernel(y2a, y2b, src_s, dst_s)
    return _p3(s2a, s2b, r2, da, db)


# speedup vs baseline: 1.1654x; 1.1654x over previous
import functools

import jax
import jax.numpy as jnp
from jax import lax
from jax.experimental import pallas as pl
from jax.experimental.pallas import tpu as pltpu
from jax.experimental.pallas import tpu_sc as plsc

N = 10000
E = 160000
D = 256
H = 128
NC = 2
NS = 16
K = 128
E_PAD = 163840
R_SH = 10112

_EPT_S = E_PAD // NS
_EPT_D = E_PAD // (NC * NS)

_mesh = plsc.VectorSubcoreMesh(
    core_axis_name="c", subcore_axis_name="s", num_cores=NC, num_subcores=NS)

def _memset2d(ref, rows, cols):
    zero16 = jnp.zeros((16,), jnp.float32)

    def body(i, _):
        for j in range(cols // 16):
            ref[i, pl.ds(j * 16, 16)] = zero16
        return 0
    lax.fori_loop(0, rows, body, 0, unroll=False)


def _zero_shared_slice(zbuf, shared, base, rows, bufrows=K):
    full, rem = rows // bufrows, rows % bufrows
    for i in range(full):
        pltpu.sync_copy(zbuf, shared.at[pl.ds(base + i * bufrows, bufrows)])
    if rem:
        pltpu.sync_copy(zbuf.at[pl.ds(0, rem)],
                        shared.at[pl.ds(base + full * bufrows, rem)])


@functools.partial(
    pl.kernel,
    out_type=[jax.ShapeDtypeStruct((N, H), jnp.float32),
              jax.ShapeDtypeStruct((N, H), jnp.float32)],
    mesh=_mesh,
    scratch_types=[
        pltpu.VMEM((K,), jnp.int32),
        pltpu.VMEM((K, H), jnp.float32),
        pltpu.VMEM_SHARED((R_SH, H), jnp.float32),
    ],
)
def _deg_kernel(dst_hbm, da_out, db_out, dst_v, ones_v, deg_sh):
    c = lax.axis_index("c")
    s = lax.axis_index("s")
    wid = s * NC + c

    _memset2d(ones_v, K, H)
    _zero_shared_slice(ones_v, deg_sh, s * (R_SH // NS), R_SH // NS)
    plsc.subcore_barrier()

    e0 = jnp.where(lax.iota(jnp.int32, 16) == 0,
                   jnp.float32(1.0), jnp.float32(0.0))

    def init_body(i, _):
        ones_v[i, pl.ds(0, 16)] = e0
        return 0
    lax.fori_loop(0, K, init_body, 0, unroll=False)

    def chunk(k, _):
        off = wid * _EPT_D + k * K
        pltpu.sync_copy(dst_hbm.at[pl.ds(off, K)], dst_v)
        pltpu.sync_copy(ones_v, deg_sh.at[dst_v], add=True)
        return 0
    lax.fori_loop(0, _EPT_D // K, chunk, 0, unroll=False)

    plsc.subcore_barrier()
    span = R_SH // NS
    last = N - span * (NS - 1)

    def copy_out(out_ref):
        @pl.when(s < NS - 1)
        def _():
            pltpu.sync_copy(deg_sh.at[pl.ds(s * span, span)],
                            out_ref.at[pl.ds(s * span, span)])

        @pl.when(s == NS - 1)
        def _():
            pltpu.sync_copy(deg_sh.at[pl.ds((NS - 1) * span, last)],
                            out_ref.at[pl.ds((NS - 1) * span, last)])

    @pl.when(c == 0)
    def _():
        copy_out(da_out)

    @pl.when(c == 1)
    def _():
        copy_out(db_out)


_CH = _EPT_S // K
_CHH = _CH // 2


@functools.partial(
    pl.kernel,
    out_type=[jax.ShapeDtypeStruct((N, H), jnp.float32),
              jax.ShapeDtypeStruct((N, H), jnp.float32)],
    mesh=_mesh,
    scratch_types=[
        pltpu.VMEM((_CHH, K), jnp.int32),
        pltpu.VMEM((_CHH, K), jnp.int32),
        pltpu.VMEM((K, H), jnp.float32),
        pltpu.VMEM((K, H), jnp.float32),
        pltpu.VMEM_SHARED((R_SH, H), jnp.float32),
        pltpu.SemaphoreType.DMA,
        pltpu.SemaphoreType.DMA,
    ],
)
def _seg_kernel(ya_hbm, yb_hbm, src_hbm, dst_hbm, sa_out, sb_out,
                src_half, dst_half, rows0, rows1, s_sh, sem0, sem1):
    c = lax.axis_index("c")
    s = lax.axis_index("s")

    _memset2d(rows0, K, H)
    _zero_shared_slice(rows0, s_sh, s * (R_SH // NS), R_SH // NS)
    plsc.subcore_barrier()

    def main(y_hbm):
        def gather(k, rows, sem):
            pltpu.async_copy(y_hbm.at[src_half.at[k]], rows, sem)

        def gwait(rows, sem):
            pltpu.make_async_copy(y_hbm.at[src_half.at[0]], rows, sem).wait()

        def scat(k, rows):
            pltpu.sync_copy(rows, s_sh.at[dst_half.at[k]], add=True)

        for phase in range(2):
            pltpu.sync_copy(
                src_hbm.at[pl.ds(s * _CH + phase * _CHH, _CHH)], src_half)
            pltpu.sync_copy(
                dst_hbm.at[pl.ds(s * _CH + phase * _CHH, _CHH)], dst_half)

            gather(0, rows0, sem0)
            gather(1, rows1, sem1)

            def body(i, _):
                k0 = i * 2
                gwait(rows0, sem0)
                scat(k0, rows0)

                @pl.when(k0 + 2 < _CHH)
                def _():
                    gather(k0 + 2, rows0, sem0)

                gwait(rows1, sem1)
                scat(k0 + 1, rows1)

                @pl.when(k0 + 3 < _CHH)
                def _():
                    gather(k0 + 3, rows1, sem1)
                return 0
            lax.fori_loop(0, _CHH // 2, body, 0, unroll=False)

    @pl.when(c == 0)
    def _():
        main(ya_hbm)

    @pl.when(c == 1)
    def _():
        main(yb_hbm)

    plsc.subcore_barrier()

    span = R_SH // NS
    last = N - span * (NS - 1)

    def copy_out(out_ref):
        @pl.when(s < NS - 1)
        def _():
            pltpu.sync_copy(s_sh.at[pl.ds(s * span, span)],
                            out_ref.at[pl.ds(s * span, span)])

        @pl.when(s == NS - 1)
        def _():
            pltpu.sync_copy(s_sh.at[pl.ds((NS - 1) * span, last)],
                            out_ref.at[pl.ds((NS - 1) * span, last)])

    @pl.when(c == 0)
    def _():
        copy_out(sa_out)

    @pl.when(c == 1)
    def _():
        copy_out(sb_out)


_RB = 1000
_GRID = N // _RB


def _dinv_of(da_blk, db_blk):
    deg = jnp.sum(da_blk, axis=1) + jnp.sum(db_blk, axis=1) + 1.0
    return lax.rsqrt(deg)[:, None]


def _p1_body(x_ref, w_ref, b_ref, da_ref, db_ref, ya_ref, yb_ref, r_ref):
    dinv = _dinv_of(da_ref[...], db_ref[...])
    x = x_ref[...]
    y = jnp.dot(x, w_ref[...], preferred_element_type=jnp.float32)
    yp = y * dinv
    ya_ref[...] = yp[:, :H]
    yb_ref[...] = yp[:, H:]
    r_ref[...] = y * (dinv * dinv) + b_ref[...] + x


def _p2_body(sa_ref, sb_ref, r1_ref, x_ref, w_ref, b_ref, da_ref, db_ref,
             ya_ref, yb_ref, r_ref):
    dinv = _dinv_of(da_ref[...], db_ref[...])
    h = jnp.concatenate([sa_ref[...] * dinv, sb_ref[...] * dinv],
                        axis=1) + r1_ref[...]
    y = jnp.dot(h, w_ref[...], preferred_element_type=jnp.float32)
    yp = y * dinv
    ya_ref[...] = yp[:, :H]
    yb_ref[...] = yp[:, H:]
    r_ref[...] = y * (dinv * dinv) + b_ref[...] + x_ref[...]


def _p3_body(sa_ref, sb_ref, r2_ref, da_ref, db_ref, o_ref):
    dinv = _dinv_of(da_ref[...], db_ref[...])
    o_ref[...] = jnp.concatenate([sa_ref[...] * dinv, sb_ref[...] * dinv],
                                 axis=1) + r2_ref[...]


def _row_spec(w):
    return pl.BlockSpec((_RB, w), lambda i: (i, 0))


_FULL_W = pl.BlockSpec((D, D), lambda i: (0, 0))
_FULL_B = pl.BlockSpec((1, D), lambda i: (0, 0))


def _p1(x, W1, b1r, da, db):
    return pl.pallas_call(
        _p1_body, grid=(_GRID,),
        in_specs=[_row_spec(D), _FULL_W, _FULL_B, _row_spec(H), _row_spec(H)],
        out_specs=[_row_spec(H), _row_spec(H), _row_spec(D)],
        out_shape=[jax.ShapeDtypeStruct((N, H), jnp.float32),
                   jax.ShapeDtypeStruct((N, H), jnp.float32),
                   jax.ShapeDtypeStruct((N, D), jnp.float32)],
    )(x, W1, b1r, da, db)


def _p2(sa, sb, r1, x, W2, b2r, da, db):
    return pl.pallas_call(
        _p2_body, grid=(_GRID,),
        in_specs=[_row_spec(H), _row_spec(H), _row_spec(D), _row_spec(D),
                  _FULL_W, _FULL_B, _row_spec(H), _row_spec(H)],
        out_specs=[_row_spec(H), _row_spec(H), _row_spec(D)],
        out_shape=[jax.ShapeDtypeStruct((N, H), jnp.float32),
                   jax.ShapeDtypeStruct((N, H), jnp.float32),
                   jax.ShapeDtypeStruct((N, D), jnp.float32)],
    )(sa, sb, r1, x, W2, b2r, da, db)


def _p3(sa, sb, r2, da, db):
    return pl.pallas_call(
        _p3_body, grid=(_GRID,),
        in_specs=[_row_spec(H), _row_spec(H), _row_spec(D),
                  _row_spec(H), _row_spec(H)],
        out_specs=_row_spec(D),
        out_shape=jax.ShapeDtypeStruct((N, D), jnp.float32),
    )(sa, sb, r2, da, db)


def kernel(x, edges, W1, b1, W2, b2):
    e = edges.astype(jnp.int32)
    pad = E_PAD - E
    src = jnp.concatenate([e[:, 0], jnp.zeros((pad,), jnp.int32)])
    dst = jnp.concatenate([e[:, 1], jnp.full((pad,), N, jnp.int32)])

    src2d = src.reshape(E_PAD // K, K)
    dst2d = dst.reshape(E_PAD // K, K)

    da, db = _deg_kernel(dst)
    ya, yb, r1 = _p1(x, W1, b1.reshape(1, D), da, db)
    sa, sb = _seg_kernel(ya, yb, src2d, dst2d)
    y2a, y2b, r2 = _p2(sa, sb, r1, x, W2, b2.reshape(1, D), da, db)
    s2a, s2b = _seg_kernel(y2a, y2b, src2d, dst2d)
    return _p3(s2a, s2b, r2, da, db)
